# baseline (speedup 1.0000x reference)
import functools

import jax
import jax.numpy as jnp
from jax import lax
from jax.experimental import pallas as pl
from jax.experimental.pallas import tpu as pltpu
from jax.experimental.pallas import tpu_sc as plsc

N = 10000
D = 128
DH = D // 2
E = 320000
NC = 2
NS = 16
L = 16

CHUNK = 128
KT = 160
EPT = KT * CHUNK
E_PAD = EPT * NS
R = 10240
RPS = R // NS
BR = 1024

_mesh = plsc.VectorSubcoreMesh(
    core_axis_name="c", subcore_axis_name="s", num_cores=NC, num_subcores=NS)
_sc_params = pltpu.CompilerParams(needs_layout_passes=False)
_sc_params_untiled = pltpu.CompilerParams(
    needs_layout_passes=False, use_tc_tiling_on_sc=False)


@functools.partial(
    pl.kernel,
    out_type=jax.ShapeDtypeStruct((R,), jnp.float32),
    mesh=_mesh,
    scratch_types=[
        pltpu.VMEM((E_PAD // NS,), jnp.int32),
        pltpu.VMEM((R,), jnp.float32),
        pltpu.VMEM((NS, RPS), jnp.float32),
        pltpu.VMEM((RPS,), jnp.float32),
        pltpu.VMEM_SHARED((NS, R), jnp.float32),
    ],
    compiler_params=_sc_params,
)
def _deg_kernel(dst_hbm, dis_hbm, dst_v, hist_v, part_v, stage_v, acc_sh):
    c = lax.axis_index("c")
    s = lax.axis_index("s")
    zeros = jnp.zeros((L,), jnp.float32)
    ones = jnp.ones((L,), jnp.float32)

    @pl.when(c == 0)
    def _():
        pltpu.sync_copy(dst_hbm.at[s], dst_v)

        def zh(i, _):
            hist_v[pl.ds(i * L, L)] = zeros
            return ()
        lax.fori_loop(0, R // L, zh, ())

        def hb(i, _):
            idx = dst_v[pl.ds(i * L, L)]
            plsc.addupdate_scatter(hist_v, [idx], ones)
            return ()
        lax.fori_loop(0, E_PAD // NS // L, hb, ())

        pltpu.sync_copy(hist_v, acc_sh.at[s])
        plsc.subcore_barrier()

        row0 = s * RPS
        for t in range(NS):
            pltpu.sync_copy(acc_sh.at[t, pl.ds(row0, RPS)], part_v.at[t])

        def rb(i, _):
            sl = pl.ds(i * L, L)
            d = part_v[0, sl] + 1.0
            for t in range(1, NS):
                d = d + part_v[t, sl]
            xi = lax.bitcast_convert_type(d, jnp.int32)
            yi = jnp.int32(0x5F3759DF) - (xi >> 1)
            y = lax.bitcast_convert_type(yi, jnp.float32)
            y = y * (1.5 - 0.5 * d * y * y)
            y = y * (1.5 - 0.5 * d * y * y)
            y = y * (1.5 - 0.5 * d * y * y)
            stage_v[sl] = y
            return ()
        lax.fori_loop(0, RPS // L, rb, ())

        pltpu.sync_copy(stage_v, dis_hbm.at[pl.ds(row0, RPS)])


@functools.partial(
    pl.kernel,
    out_type=jax.ShapeDtypeStruct((NC, R, DH), jnp.float32),
    mesh=_mesh,
    scratch_types=[
        pltpu.VMEM((KT, CHUNK), jnp.int32),
        pltpu.VMEM((KT, CHUNK), jnp.int32),
        pltpu.VMEM((4, CHUNK, DH), jnp.float32),
        pltpu.VMEM_SHARED((R, DH), jnp.float32),
        pltpu.SemaphoreType.DMA,
        pltpu.SemaphoreType.DMA,
        pltpu.SemaphoreType.DMA,
        pltpu.SemaphoreType.DMA,
        pltpu.SemaphoreType.DMA,
        pltpu.SemaphoreType.DMA,
        pltpu.SemaphoreType.DMA,
        pltpu.SemaphoreType.DMA,
    ],
    compiler_params=_sc_params_untiled,
)
def _agg_kernel(src_hbm, dst_hbm, table_hbm, out_hbm,
                src_v, dst_v, rows_v, acc_sh,
                g0, g1, g2, g3, s0, s1, s2, s3):
    c = lax.axis_index("c")
    s = lax.axis_index("s")
    gsem = (g0, g1, g2, g3)
    ssem = (s0, s1, s2, s3)
    zeros = jnp.zeros((L,), jnp.float32)

    pltpu.sync_copy(src_hbm.at[s], src_v)
    pltpu.sync_copy(dst_hbm.at[s], dst_v)

    def zb(i, _):
        r = i // (DH // L)
        u = i % (DH // L)
        rows_v[0, r, pl.ds(u * L, L)] = zeros
        return ()
    lax.fori_loop(0, CHUNK * (DH // L), zb, ())
    row0 = s * RPS
    for j in range(RPS // CHUNK):
        pltpu.sync_copy(rows_v.at[0], acc_sh.at[pl.ds(row0 + j * CHUNK, CHUNK)])

    tbl = table_hbm.at[c]
    for p in range(2):
        pltpu.async_copy(tbl.at[src_v.at[p]], rows_v.at[p], gsem[p])
    plsc.subcore_barrier()

    def body(t, _):
        j0 = 4 * t
        for k in range(4):
            j = j0 + k
            b = k
            bn = (k + 2) % 4
            pltpu.make_async_copy(tbl.at[src_v.at[j]], rows_v.at[b], gsem[b]).wait()
            pltpu.async_copy(rows_v.at[b], acc_sh.at[dst_v.at[j]], ssem[b], add=True)

            @pl.when(j >= 2)
            def _():
                pltpu.make_async_copy(
                    rows_v.at[bn], acc_sh.at[dst_v.at[j]], ssem[bn]).wait()

            @pl.when(j + 2 < KT)
            def _():
                pltpu.async_copy(tbl.at[src_v.at[j + 2]], rows_v.at[bn], gsem[bn])
        return ()
    lax.fori_loop(0, KT // 4, body, ())

    pltpu.make_async_copy(rows_v.at[2], acc_sh.at[dst_v.at[KT - 2]], ssem[2]).wait()
    pltpu.make_async_copy(rows_v.at[3], acc_sh.at[dst_v.at[KT - 1]], ssem[3]).wait()

    plsc.subcore_barrier()
    pltpu.sync_copy(acc_sh.at[pl.ds(row0, RPS)], out_hbm.at[c, pl.ds(row0, RPS)])


def _split(h):
    return h[:, :DH], h[:, DH:]


def _tc1a_body(x_ref, w_ref, out_ref):
    i = pl.program_id(0)
    h = jnp.dot(x_ref[...], w_ref[...], preferred_element_type=jnp.float32)
    row = i * BR + jax.lax.broadcasted_iota(jnp.int32, (BR, 1), 0)
    out_ref[...] = jnp.where(row < N, h, 0.0)


def _tc1b_body(h_ref, dis_ref, out_ref):
    h = dis_ref[...] * h_ref[...]
    lo, hi = _split(h)
    out_ref[0] = lo
    out_ref[1] = hi


def _tc2_body(agg_ref, tbl_ref, dis_ref, b_ref, w_ref, out_ref):
    dis = dis_ref[...]
    a = jnp.concatenate([agg_ref[0], agg_ref[1]], axis=1)
    t = jnp.concatenate([tbl_ref[0], tbl_ref[1]], axis=1)
    v = dis * (a + t) + b_ref[...]
    z = jnp.maximum(v, 0.0)
    h = dis * jnp.dot(z, w_ref[...], preferred_element_type=jnp.float32)
    lo, hi = _split(h)
    out_ref[0] = lo
    out_ref[1] = hi


def _tc3_body(agg_ref, tbl_ref, dis_ref, b_ref, out_ref):
    a = jnp.concatenate([agg_ref[0], agg_ref[1]], axis=1)
    t = jnp.concatenate([tbl_ref[0], tbl_ref[1]], axis=1)
    v = dis_ref[...] * (a + t) + b_ref[...]
    m = jnp.max(v, axis=1, keepdims=True)
    e = jnp.exp(v - m)
    lse = jnp.log(jnp.sum(e, axis=1, keepdims=True))
    out_ref[...] = v - m - lse


def _half_spec():
    return pl.BlockSpec((NC, BR, DH), lambda i: (0, i, 0))


_tc1a = pl.pallas_call(
    _tc1a_body,
    grid=(R // BR,),
    in_specs=[pl.BlockSpec((BR, D), lambda i: (i, 0)),
              pl.BlockSpec((D, D), lambda i: (0, 0))],
    out_specs=pl.BlockSpec((BR, D), lambda i: (i, 0)),
    out_shape=jax.ShapeDtypeStruct((R, D), jnp.float32),
)

_tc1b = pl.pallas_call(
    _tc1b_body,
    grid=(R // BR,),
    in_specs=[pl.BlockSpec((BR, D), lambda i: (i, 0)),
              pl.BlockSpec((BR, 1), lambda i: (i, 0))],
    out_specs=_half_spec(),
    out_shape=jax.ShapeDtypeStruct((NC, R, DH), jnp.float32),
)

_tc2 = pl.pallas_call(
    _tc2_body,
    grid=(R // BR,),
    in_specs=[_half_spec(),
              _half_spec(),
              pl.BlockSpec((BR, 1), lambda i: (i, 0)),
              pl.BlockSpec((1, D), lambda i: (0, 0)),
              pl.BlockSpec((D, D), lambda i: (0, 0))],
    out_specs=_half_spec(),
    out_shape=jax.ShapeDtypeStruct((NC, R, DH), jnp.float32),
)

_tc3 = pl.pallas_call(
    _tc3_body,
    grid=(R // BR,),
    in_specs=[_half_spec(),
              _half_spec(),
              pl.BlockSpec((BR, 1), lambda i: (i, 0)),
              pl.BlockSpec((1, D), lambda i: (0, 0))],
    out_specs=pl.BlockSpec((BR, D), lambda i: (i, 0)),
    out_shape=jax.ShapeDtypeStruct((N, D), jnp.float32),
)


def kernel(x, edge_index, W1, b1, W2, b2):
    src = edge_index[0].astype(jnp.int32)
    dst = edge_index[1].astype(jnp.int32)
    pad = N + (jnp.arange(E_PAD - E, dtype=jnp.int32) % (R - N))
    src_p = jnp.concatenate([src, pad]).reshape(NS, KT, CHUNK)
    dst_p = jnp.concatenate([dst, pad]).reshape(NS, KT, CHUNK)
    dst_rows = dst_p.reshape(NS, E_PAD // NS)

    b1r = b1.reshape(1, D)
    b2r = b2.reshape(1, D)

    h1 = _tc1a(x, W1)
    dis = _deg_kernel(dst_rows)
    dis_col = dis.reshape(R, 1)

    table1 = _tc1b(h1, dis_col)
    agg1 = _agg_kernel(src_p, dst_p, table1)
    table2 = _tc2(agg1, table1, dis_col, b1r, W2)
    agg2 = _agg_kernel(src_p, dst_p, table2)
    return _tc3(agg2, table2, dis_col, b2r)


# speedup vs baseline: 1.1619x; 1.1619x over previous
import functools

import jax
import jax.numpy as jnp
from jax import lax
from jax.experimental import pallas as pl
from jax.experimental.pallas import tpu as pltpu
from jax.experimental.pallas import tpu_sc as plsc

N = 10000
D = 128
DH = D // 2
E = 320000
NC = 2
NS = 16
L = 16

CHUNK = 128
KT = 160
EPT = KT * CHUNK
E_PAD = EPT * NS
R = 10240
RPS = R // NS
BR = 1024

_mesh = plsc.VectorSubcoreMesh(
    core_axis_name="c", subcore_axis_name="s", num_cores=NC, num_subcores=NS)
_sc_params = pltpu.CompilerParams(needs_layout_passes=False)
_sc_params_untiled = pltpu.CompilerParams(
    needs_layout_passes=False, use_tc_tiling_on_sc=False)


@functools.partial(
    pl.kernel,
    out_type=jax.ShapeDtypeStruct((R,), jnp.float32),
    mesh=_mesh,
    scratch_types=[
        pltpu.VMEM((E_PAD // NS,), jnp.int32),
        pltpu.VMEM((R,), jnp.float32),
        pltpu.VMEM((NS, RPS), jnp.float32),
        pltpu.VMEM((RPS,), jnp.float32),
        pltpu.VMEM_SHARED((NS, R), jnp.float32),
    ],
    compiler_params=_sc_params,
)
def _deg_kernel(dst_hbm, dis_hbm, dst_v, hist_v, part_v, stage_v, acc_sh):
    c = lax.axis_index("c")
    s = lax.axis_index("s")
    zeros = jnp.zeros((L,), jnp.float32)
    ones = jnp.ones((L,), jnp.float32)

    @pl.when(c == 0)
    def _():
        pltpu.sync_copy(dst_hbm.at[s], dst_v)

        def zh(i, _):
            hist_v[pl.ds(i * L, L)] = zeros
            return ()
        lax.fori_loop(0, R // L, zh, ())

        def hb(i, _):
            idx = dst_v[pl.ds(i * L, L)]
            plsc.addupdate_scatter(hist_v, [idx], ones)
            return ()
        lax.fori_loop(0, E_PAD // NS // L, hb, ())

        pltpu.sync_copy(hist_v, acc_sh.at[s])
        plsc.subcore_barrier()

        row0 = s * RPS
        for t in range(NS):
            pltpu.sync_copy(acc_sh.at[t, pl.ds(row0, RPS)], part_v.at[t])

        def rb(i, _):
            sl = pl.ds(i * L, L)
            d = part_v[0, sl] + 1.0
            for t in range(1, NS):
                d = d + part_v[t, sl]
            xi = lax.bitcast_convert_type(d, jnp.int32)
            yi = jnp.int32(0x5F3759DF) - (xi >> 1)
            y = lax.bitcast_convert_type(yi, jnp.float32)
            y = y * (1.5 - 0.5 * d * y * y)
            y = y * (1.5 - 0.5 * d * y * y)
            y = y * (1.5 - 0.5 * d * y * y)
            stage_v[sl] = y
            return ()
        lax.fori_loop(0, RPS // L, rb, ())

        pltpu.sync_copy(stage_v, dis_hbm.at[pl.ds(row0, RPS)])


@functools.partial(
    pl.kernel,
    out_type=jax.ShapeDtypeStruct((NC, R, DH), jnp.float32),
    mesh=_mesh,
    scratch_types=[
        pltpu.VMEM((KT, CHUNK), jnp.int32),
        pltpu.VMEM((KT, CHUNK), jnp.int32),
        pltpu.VMEM((4, CHUNK, DH), jnp.float32),
        pltpu.VMEM_SHARED((R, DH), jnp.float32),
        pltpu.SemaphoreType.DMA,
        pltpu.SemaphoreType.DMA,
        pltpu.SemaphoreType.DMA,
        pltpu.SemaphoreType.DMA,
        pltpu.SemaphoreType.DMA,
        pltpu.SemaphoreType.DMA,
        pltpu.SemaphoreType.DMA,
        pltpu.SemaphoreType.DMA,
    ],
    compiler_params=_sc_params_untiled,
)
def _agg_kernel(src_hbm, dst_hbm, table_hbm, out_hbm,
                src_v, dst_v, rows_v, acc_sh,
                g0, g1, g2, g3, s0, s1, s2, s3):
    c = lax.axis_index("c")
    s = lax.axis_index("s")
    gsem = (g0, g1, g2, g3)
    ssem = (s0, s1, s2, s3)
    zeros = jnp.zeros((L,), jnp.float32)

    pltpu.sync_copy(src_hbm.at[s], src_v)
    pltpu.sync_copy(dst_hbm.at[s], dst_v)

    def zb(i, _):
        r = i // (DH // L)
        u = i % (DH // L)
        rows_v[0, r, pl.ds(u * L, L)] = zeros
        return ()
    lax.fori_loop(0, CHUNK * (DH // L), zb, ())
    row0 = s * RPS
    for j in range(RPS // CHUNK):
        pltpu.sync_copy(rows_v.at[0], acc_sh.at[pl.ds(row0 + j * CHUNK, CHUNK)])

    tbl = table_hbm.at[c]
    for p in range(4):
        pltpu.async_copy(tbl.at[src_v.at[p]], rows_v.at[p], gsem[p])
    plsc.subcore_barrier()

    def body(t, _):
        j0 = 4 * t
        for p in range(4):
            j = j0 + p
            pltpu.make_async_copy(tbl.at[src_v.at[j]], rows_v.at[p], gsem[p]).wait()
            pltpu.sync_copy(rows_v.at[p], acc_sh.at[dst_v.at[j]], add=True)

            @pl.when(j + 4 < KT)
            def _():
                pltpu.async_copy(tbl.at[src_v.at[j + 4]], rows_v.at[p], gsem[p])
        return ()
    lax.fori_loop(0, KT // 4, body, ())

    plsc.subcore_barrier()
    pltpu.sync_copy(acc_sh.at[pl.ds(row0, RPS)], out_hbm.at[c, pl.ds(row0, RPS)])


def _split(h):
    return h[:, :DH], h[:, DH:]


def _tc1a_body(x_ref, w_ref, out_ref):
    i = pl.program_id(0)
    h = jnp.dot(x_ref[...], w_ref[...], preferred_element_type=jnp.float32)
    row = i * BR + jax.lax.broadcasted_iota(jnp.int32, (BR, 1), 0)
    out_ref[...] = jnp.where(row < N, h, 0.0)


def _tc1b_body(h_ref, dis_ref, out_ref):
    h = dis_ref[...] * h_ref[...]
    lo, hi = _split(h)
    out_ref[0] = lo
    out_ref[1] = hi


def _tc2_body(agg_ref, tbl_ref, dis_ref, b_ref, w_ref, out_ref):
    dis = dis_ref[...]
    a = jnp.concatenate([agg_ref[0], agg_ref[1]], axis=1)
    t = jnp.concatenate([tbl_ref[0], tbl_ref[1]], axis=1)
    v = dis * (a + t) + b_ref[...]
    z = jnp.maximum(v, 0.0)
    h = dis * jnp.dot(z, w_ref[...], preferred_element_type=jnp.float32)
    lo, hi = _split(h)
    out_ref[0] = lo
    out_ref[1] = hi


def _tc3_body(agg_ref, tbl_ref, dis_ref, b_ref, out_ref):
    a = jnp.concatenate([agg_ref[0], agg_ref[1]], axis=1)
    t = jnp.concatenate([tbl_ref[0], tbl_ref[1]], axis=1)
    v = dis_ref[...] * (a + t) + b_ref[...]
    m = jnp.max(v, axis=1, keepdims=True)
    e = jnp.exp(v - m)
    lse = jnp.log(jnp.sum(e, axis=1, keepdims=True))
    out_ref[...] = v - m - lse


def _half_spec():
    return pl.BlockSpec((NC, BR, DH), lambda i: (0, i, 0))


_tc1a = pl.pallas_call(
    _tc1a_body,
    grid=(R // BR,),
    in_specs=[pl.BlockSpec((BR, D), lambda i: (i, 0)),
              pl.BlockSpec((D, D), lambda i: (0, 0))],
    out_specs=pl.BlockSpec((BR, D), lambda i: (i, 0)),
    out_shape=jax.ShapeDtypeStruct((R, D), jnp.float32),
)

_tc1b = pl.pallas_call(
    _tc1b_body,
    grid=(R // BR,),
    in_specs=[pl.BlockSpec((BR, D), lambda i: (i, 0)),
              pl.BlockSpec((BR, 1), lambda i: (i, 0))],
    out_specs=_half_spec(),
    out_shape=jax.ShapeDtypeStruct((NC, R, DH), jnp.float32),
)

_tc2 = pl.pallas_call(
    _tc2_body,
    grid=(R // BR,),
    in_specs=[_half_spec(),
              _half_spec(),
              pl.BlockSpec((BR, 1), lambda i: (i, 0)),
              pl.BlockSpec((1, D), lambda i: (0, 0)),
              pl.BlockSpec((D, D), lambda i: (0, 0))],
    out_specs=_half_spec(),
    out_shape=jax.ShapeDtypeStruct((NC, R, DH), jnp.float32),
)

_tc3 = pl.pallas_call(
    _tc3_body,
    grid=(R // BR,),
    in_specs=[_half_spec(),
              _half_spec(),
              pl.BlockSpec((BR, 1), lambda i: (i, 0)),
              pl.BlockSpec((1, D), lambda i: (0, 0))],
    out_specs=pl.BlockSpec((BR, D), lambda i: (i, 0)),
    out_shape=jax.ShapeDtypeStruct((N, D), jnp.float32),
)


def kernel(x, edge_index, W1, b1, W2, b2):
    src = edge_index[0].astype(jnp.int32)
    dst = edge_index[1].astype(jnp.int32)
    pad = N + (jnp.arange(E_PAD - E, dtype=jnp.int32) % (R - N))
    src_p = jnp.concatenate([src, pad]).reshape(NS, KT, CHUNK)
    dst_p = jnp.concatenate([dst, pad]).reshape(NS, KT, CHUNK)
    dst_rows = dst_p.reshape(NS, E_PAD // NS)

    b1r = b1.reshape(1, D)
    b2r = b2.reshape(1, D)

    h1 = _tc1a(x, W1)
    dis = _deg_kernel(dst_rows)
    dis_col = dis.reshape(R, 1)

    table1 = _tc1b(h1, dis_col)
    agg1 = _agg_kernel(src_p, dst_p, table1)
    table2 = _tc2(agg1, table1, dis_col, b1r, W2)
    agg2 = _agg_kernel(src_p, dst_p, table2)
    return _tc3(agg2, table2, dis_col, b2r)
